# baseline (speedup 1.0000x reference)
import functools

import jax
import jax.numpy as jnp
from jax import lax
from jax.experimental import pallas as pl
from jax.experimental.pallas import tpu as pltpu
from jax.experimental.pallas import tpu_sc as plsc


_ROWS = 8
_NR = 4
_PF = 2


def _make_sc_kernel(b, s, d):
    info = plsc.get_sparse_core_info()
    nc, ns, lanes = info.num_cores, info.num_subcores, info.num_lanes
    nw = nc * ns
    rows_w = s // nw
    n_slabs = rows_w // _ROWS
    nsteps = n_slabs * b
    groups = d // lanes
    assert s % nw == 0 and rows_w % _ROWS == 0 and d % lanes == 0
    assert _NR % b == 0 or b % _NR == 0
    assert (nsteps - 2 * _PF) % _NR == 0
    mesh = plsc.VectorSubcoreMesh(core_axis_name="c", subcore_axis_name="s")

    @functools.partial(
        pl.kernel,
        mesh=mesh,
        out_type=jax.ShapeDtypeStruct((b, s, d), jnp.float32),
        scratch_types=(
            [pltpu.VMEM((rows_w, d), jnp.float32)]
            + [pltpu.VMEM((_ROWS, d), jnp.float32) for _ in range(_NR)]
            + [pltpu.SemaphoreType.DMA for _ in range(3)]
        ),
    )
    def k(x_hbm, pos_hbm, out_hbm, pbuf, *bufs_and_sems):
        xbufs = bufs_and_sems[:_NR]
        psem, lsem, ssem = bufs_and_sems[_NR:]
        wid = lax.axis_index("s") * nc + lax.axis_index("c")
        base = wid * rows_w

        def issue_load(c, bb, slot):
            pltpu.async_copy(
                x_hbm.at[bb, pl.ds(base + c * _ROWS, _ROWS)], xbufs[slot],
                lsem)

        def wait_load(slot):
            pltpu.make_async_copy(
                x_hbm.at[0, pl.ds(0, _ROWS)], xbufs[slot], lsem).wait()

        def issue_store(c, bb, slot):
            pltpu.async_copy(
                xbufs[slot], out_hbm.at[bb, pl.ds(base + c * _ROWS, _ROWS)],
                ssem)

        def wait_store(slot):
            pltpu.make_async_copy(
                xbufs[slot], out_hbm.at[0, pl.ds(0, _ROWS)], ssem).wait()

        def compute(c, slot):
            buf = xbufs[slot]

            def row_body(r, _, buf=buf, c=c):
                def add_body(i, _2, r=r, buf=buf, c=c):
                    sl = pl.ds(i * lanes, lanes)
                    plsc.addupdate(buf.at[r, sl], pbuf[c * _ROWS + r, sl])
                    return _2

                lax.fori_loop(0, groups, add_body, 0, unroll=4)
                return _

            lax.fori_loop(0, _ROWS, row_body, 0)

        pcp = pltpu.async_copy(pos_hbm.at[pl.ds(base, rows_w)], pbuf, psem)
        for j in range(_PF):
            issue_load(j // b, j % b, j % _NR)
        pcp.wait()

        for j in range(_PF):
            wait_load(j % _NR)
            compute(j // b, j % _NR)
            issue_load((j + _PF) // b, (j + _PF) % b, (j + _PF) % _NR)
            issue_store(j // b, j % b, j % _NR)

        n_outer = (nsteps - 2 * _PF) // _NR

        def outer_body(j0, _):
            for kk in range(_NR):
                slot = (_PF + kk) % _NR
                bb = (_PF + kk) % b
                jj = _PF + j0 * _NR + kk
                c = jj // b
                wait_load(slot)
                compute(c, slot)
                nslot = (slot + _PF) % _NR
                wait_store(nslot)
                issue_load((jj + _PF) // b, (bb + _PF) % b, nslot)
                issue_store(c, bb, slot)
            return _

        lax.fori_loop(0, n_outer, outer_body, 0)

        for j in range(nsteps - _PF, nsteps):
            slot = j % _NR
            wait_load(slot)
            compute(j // b, slot)
            wait_store((slot + _PF) % _NR)
            issue_store(j // b, j % b, slot)

        for j in range(nsteps - _PF, nsteps):
            wait_store(j % _NR)

    return k


def kernel(x, pos_table):
    b, s, d = x.shape
    k = _make_sc_kernel(b, s, d)
    return k(x, pos_table[:s])


# speedup vs baseline: 1.6512x; 1.6512x over previous
import functools

import jax
import jax.numpy as jnp
from jax import lax
from jax.experimental import pallas as pl
from jax.experimental.pallas import tpu as pltpu
from jax.experimental.pallas import tpu_sc as plsc


_ROWS = 8
_DEPTH = 3


def _make_sc_kernel(b, s, d):
    info = plsc.get_sparse_core_info()
    nc, ns, lanes = info.num_cores, info.num_subcores, info.num_lanes
    nw = nc * ns
    rows_w = s // nw
    assert s % nw == 0 and rows_w % _ROWS == 0
    n_slabs = rows_w // _ROWS
    groups = d // lanes
    mesh = plsc.VectorSubcoreMesh(core_axis_name="c", subcore_axis_name="s")

    @functools.partial(
        pl.kernel,
        mesh=mesh,
        out_type=jax.ShapeDtypeStruct((b, s, d), jnp.float32),
        scratch_types=(
            [pltpu.VMEM((_ROWS, d), jnp.float32)
             for _ in range(_DEPTH * (b + 1))]
            + [pltpu.SemaphoreType.DMA for _ in range(2)]
        ),
    )
    def k(x_hbm, pos_hbm, out_hbm, *bufs_and_sems):
        nbuf = _DEPTH * (b + 1)
        slots = [bufs_and_sems[i * (b + 1):(i + 1) * (b + 1)]
                 for i in range(_DEPTH)]
        lsem, ssem = bufs_and_sems[nbuf:]
        wid = lax.axis_index("s") * nc + lax.axis_index("c")
        base = wid * rows_w

        def load_slab(c):
            slot = slots[c % _DEPTH]
            r0 = base + c * _ROWS
            cps = [pltpu.async_copy(pos_hbm.at[pl.ds(r0, _ROWS)], slot[0],
                                    lsem)]
            for bb in range(b):
                cps.append(pltpu.async_copy(
                    x_hbm.at[bb, pl.ds(r0, _ROWS)], slot[1 + bb], lsem))
            return cps

        def store_slab(c):
            slot = slots[c % _DEPTH]
            r0 = base + c * _ROWS
            return [pltpu.async_copy(
                slot[1 + bb], out_hbm.at[bb, pl.ds(r0, _ROWS)], ssem)
                for bb in range(b)]

        loads = {c: load_slab(c) for c in range(min(2, n_slabs))}
        stores = {}

        for c in range(n_slabs):
            slot = slots[c % _DEPTH]
            pbuf = slot[0]
            for cp in loads.pop(c):
                cp.wait()

            def row_body(r, _, slot=slot, pbuf=pbuf):
                def add_body(i, _2, r=r, slot=slot, pbuf=pbuf):
                    sl = pl.ds(i * lanes, lanes)
                    p = pbuf[r, sl]
                    for bb in range(b):
                        plsc.addupdate(slot[1 + bb].at[r, sl], p)
                    return _2

                lax.fori_loop(0, groups, add_body, 0, unroll=4)
                return _

            lax.fori_loop(0, _ROWS, row_body, 0)
            if c - 1 in stores:
                for cp in stores.pop(c - 1):
                    cp.wait()
            if c + 2 < n_slabs:
                loads[c + 2] = load_slab(c + 2)
            stores[c] = store_slab(c)

        for c in sorted(stores):
            for cp in stores.pop(c):
                cp.wait()

    return k


def kernel(x, pos_table):
    b, s, d = x.shape
    k = _make_sc_kernel(b, s, d)
    return k(x, pos_table[:s])


# speedup vs baseline: 1.6947x; 1.0263x over previous
import functools

import jax
import jax.numpy as jnp
from jax import lax
from jax.experimental import pallas as pl
from jax.experimental.pallas import tpu as pltpu
from jax.experimental.pallas import tpu_sc as plsc


_ROWS = 4
_DEPTH = 6
_PF = 4


def _make_sc_kernel(b, s, d):
    info = plsc.get_sparse_core_info()
    nc, ns, lanes = info.num_cores, info.num_subcores, info.num_lanes
    nw = nc * ns
    rows_w = s // nw
    assert s % nw == 0 and rows_w % _ROWS == 0
    n_slabs = rows_w // _ROWS
    groups = d // lanes
    mesh = plsc.VectorSubcoreMesh(core_axis_name="c", subcore_axis_name="s")

    @functools.partial(
        pl.kernel,
        mesh=mesh,
        out_type=jax.ShapeDtypeStruct((b, s, d), jnp.float32),
        scratch_types=(
            [pltpu.VMEM((_ROWS, d), jnp.float32)
             for _ in range(_DEPTH * (b + 1))]
            + [pltpu.SemaphoreType.DMA for _ in range(2)]
        ),
    )
    def k(x_hbm, pos_hbm, out_hbm, *bufs_and_sems):
        nbuf = _DEPTH * (b + 1)
        slots = [bufs_and_sems[i * (b + 1):(i + 1) * (b + 1)]
                 for i in range(_DEPTH)]
        lsem, ssem = bufs_and_sems[nbuf:]
        wid = lax.axis_index("s") * nc + lax.axis_index("c")
        base = wid * rows_w

        def load_slab(c):
            slot = slots[c % _DEPTH]
            r0 = base + c * _ROWS
            cps = [pltpu.async_copy(pos_hbm.at[pl.ds(r0, _ROWS)], slot[0],
                                    lsem)]
            for bb in range(b):
                cps.append(pltpu.async_copy(
                    x_hbm.at[bb, pl.ds(r0, _ROWS)], slot[1 + bb], lsem))
            return cps

        def store_slab(c):
            slot = slots[c % _DEPTH]
            r0 = base + c * _ROWS
            return [pltpu.async_copy(
                slot[1 + bb], out_hbm.at[bb, pl.ds(r0, _ROWS)], ssem)
                for bb in range(b)]

        loads = {c: load_slab(c) for c in range(min(_PF, n_slabs))}
        stores = {}

        for c in range(n_slabs):
            slot = slots[c % _DEPTH]
            pbuf = slot[0]
            for cp in loads.pop(c):
                cp.wait()

            def row_body(r, _, slot=slot, pbuf=pbuf):
                def add_body(i, _2, r=r, slot=slot, pbuf=pbuf):
                    sl = pl.ds(i * lanes, lanes)
                    p = pbuf[r, sl]
                    for bb in range(b):
                        plsc.addupdate(slot[1 + bb].at[r, sl], p)
                    return _2

                lax.fori_loop(0, groups, add_body, 0, unroll=4)
                return _

            lax.fori_loop(0, _ROWS, row_body, 0)
            drain = c - (_DEPTH - _PF)
            if drain in stores:
                for cp in stores.pop(drain):
                    cp.wait()
            if c + _PF < n_slabs:
                loads[c + _PF] = load_slab(c + _PF)
            stores[c] = store_slab(c)

        for c in sorted(stores):
            for cp in stores.pop(c):
                cp.wait()

    return k


def kernel(x, pos_table):
    b, s, d = x.shape
    k = _make_sc_kernel(b, s, d)
    return k(x, pos_table[:s])
